# Initial kernel scaffold; baseline (speedup 1.0000x reference)
import jax
import jax.numpy as jnp
from jax.experimental import pallas as pl


def kernel(x, edge_index, batch, W1, b1, W2, b2, Wmu, bmu, Wlv, blv):
    raise NotImplementedError("write your pallas kernel here")


# speedup vs baseline: 22.1091x; 22.1091x over previous
import functools

import jax
import jax.numpy as jnp
from jax import lax
from jax.experimental import pallas as pl
from jax.experimental.pallas import tpu as pltpu
from jax.experimental.pallas import tpu_sc as plsc

N = 10000
E = 320000
D = 128
G = 64
L = 32

NPAD = 10240
NC = 2
NS = 16
NW = NC * NS
EPW = E // NW
CD = 80
NCHD = EPW // CD
EPT = E // NS
CA = 80
NCHT = EPT // CA
DH = D // 2
TROWS = NPAD // NS
ZR = 128
BR = 512
NB = NPAD // BR

def _deg_body(d_hbm, out, didx_all, ones_v, zb, acc):
    cid = lax.axis_index("c")
    sid = lax.axis_index("s")
    wid = cid * NS + sid

    @pl.loop(0, CD // 16)
    def _(i):
        ones_v[pl.ds(i * 16, 16)] = jnp.ones((16,), jnp.float32)

    @pl.loop(0, TROWS // 16)
    def _(i):
        zb[pl.ds(i * 16, 16)] = jnp.zeros((16,), jnp.float32)

    pltpu.sync_copy(zb, acc.at[pl.ds(sid * TROWS, TROWS)])
    pltpu.sync_copy(d_hbm.at[wid], didx_all)
    plsc.subcore_barrier()

    @pl.loop(0, NCHD)
    def _(g):
        pltpu.sync_copy(ones_v, acc.at[didx_all.at[g]], add=True)

    plsc.subcore_barrier()
    r = cid * NPAD + sid * TROWS
    pltpu.sync_copy(acc.at[pl.ds(sid * TROWS, TROWS)],
                    out.at[pl.ds(r, TROWS)])


def _agg_body(hp, s_hbm, d_hbm, out, s2v, d2v, rows0, rows1, zb,
              acc, sem0, sem1):
    cid = lax.axis_index("c")
    sid = lax.axis_index("s")

    @pl.loop(0, ZR)
    def _(i):
        for j in range(DH // 16):
            zb[i, pl.ds(j * 16, 16)] = jnp.zeros((16,), jnp.float32)

    @pl.loop(0, TROWS // ZR)
    def _(k):
        pltpu.sync_copy(zb, acc.at[pl.ds(sid * TROWS + k * ZR, ZR)])

    pltpu.sync_copy(s_hbm.at[sid], s2v)
    pltpu.sync_copy(d_hbm.at[sid], d2v)
    off = jnp.full((16,), cid * NPAD, jnp.int32)

    @pl.loop(0, NCHT)
    def _(g):
        for j in range(CA // 16):
            s2v[g, pl.ds(j * 16, 16)] = s2v[g, pl.ds(j * 16, 16)] + off

    plsc.subcore_barrier()

    pltpu.async_copy(hp.at[s2v.at[0]], rows0, sem0)

    @pl.loop(0, NCHT, step=2)
    def _(g):
        pltpu.async_copy(hp.at[s2v.at[g + 1]], rows1, sem1)
        pltpu.make_async_copy(hp.at[s2v.at[g]], rows0, sem0).wait()
        pltpu.sync_copy(rows0, acc.at[d2v.at[g]], add=True)

        @pl.when(g + 2 < NCHT)
        def _():
            pltpu.async_copy(hp.at[s2v.at[g + 2]], rows0, sem0)

        pltpu.make_async_copy(hp.at[s2v.at[g + 1]], rows1, sem1).wait()
        pltpu.sync_copy(rows1, acc.at[d2v.at[g + 1]], add=True)

    plsc.subcore_barrier()

    @pl.loop(0, TROWS // ZR)
    def _(k):
        r = sid * TROWS + k * ZR
        pltpu.sync_copy(acc.at[pl.ds(r, ZR)],
                        out.at[pl.ds(cid * NPAD + r, ZR)])


@functools.lru_cache(maxsize=None)
def _sc_kernels():
    mesh = plsc.VectorSubcoreMesh(core_axis_name="c", subcore_axis_name="s")
    params = pltpu.CompilerParams(use_tc_tiling_on_sc=False)
    deg = functools.partial(
        pl.kernel,
        out_type=jax.ShapeDtypeStruct((2 * NPAD,), jnp.float32),
        mesh=mesh,
        scratch_types=[
            pltpu.VMEM((NCHD, CD), jnp.int32),
            pltpu.VMEM((CD,), jnp.float32),
            pltpu.VMEM((TROWS,), jnp.float32),
            pltpu.VMEM_SHARED((NPAD,), jnp.float32),
        ],
        compiler_params=params,
    )(_deg_body)
    agg = functools.partial(
        pl.kernel,
        out_type=jax.ShapeDtypeStruct((2 * NPAD, DH), jnp.float32),
        mesh=mesh,
        scratch_types=[
            pltpu.VMEM((NCHT, CA), jnp.int32),
            pltpu.VMEM((NCHT, CA), jnp.int32),
            pltpu.VMEM((CA, DH), jnp.float32),
            pltpu.VMEM((CA, DH), jnp.float32),
            pltpu.VMEM((ZR, DH), jnp.float32),
            pltpu.VMEM_SHARED((NPAD, DH), jnp.float32),
            pltpu.SemaphoreType.DMA,
            pltpu.SemaphoreType.DMA,
        ],
        compiler_params=params,
    )(_agg_body)
    return deg, agg


def _dinv_body(c0_ref, c1_ref, o_ref):
    deg = 1.0 + c0_ref[...] + c1_ref[...]
    dinv = lax.rsqrt(deg)
    o_ref[...] = jnp.broadcast_to(dinv[:, :, None], (NPAD // D, D, D))


_dinv_kernel = pl.pallas_call(
    _dinv_body,
    out_shape=jax.ShapeDtypeStruct((NPAD // D, D, D), jnp.float32),
)


_STACK_SPEC = pl.BlockSpec((2, BR, DH), lambda i: (0, i, 0))
_STACK_OUT = jax.ShapeDtypeStruct((2, NPAD, DH), jnp.float32)


def _split_halves(o):
    return jnp.stack([o[:, :DH], o[:, DH:]], axis=0)


def _join_halves(s3):
    return jnp.concatenate([s3[0], s3[1]], axis=1)


def _mm_scale_body(x_ref, w_ref, dv_ref, o_ref):
    h = lax.dot_general(x_ref[...], w_ref[...], (((1,), (1,)), ((), ())),
                        preferred_element_type=jnp.float32)
    o_ref[...] = _split_halves(h * dv_ref[...])


_mm_scale_kernel = pl.pallas_call(
    _mm_scale_body,
    grid=(NB,),
    in_specs=[
        pl.BlockSpec((BR, D), lambda i: (i, 0)),
        pl.BlockSpec((D, D), lambda i: (0, 0)),
        pl.BlockSpec((BR, D), lambda i: (i, 0)),
    ],
    out_specs=_STACK_SPEC,
    out_shape=_STACK_OUT,
)


def _combine_mm_body(a_ref, h_ref, dv_ref, b_ref, w_ref, o_ref):
    dv = dv_ref[...]
    acc = _join_halves(a_ref[...] + h_ref[...])
    t = acc * dv + b_ref[...]
    h1 = jnp.maximum(t, 0.0)
    h2 = lax.dot_general(h1, w_ref[...], (((1,), (1,)), ((), ())),
                         preferred_element_type=jnp.float32)
    o_ref[...] = _split_halves(h2 * dv)


_combine_mm_kernel = pl.pallas_call(
    _combine_mm_body,
    grid=(NB,),
    in_specs=[
        _STACK_SPEC,
        _STACK_SPEC,
        pl.BlockSpec((BR, D), lambda i: (i, 0)),
        pl.BlockSpec((1, D), lambda i: (0, 0)),
        pl.BlockSpec((D, D), lambda i: (0, 0)),
    ],
    out_specs=_STACK_SPEC,
    out_shape=_STACK_OUT,
)


def _final_body(a_ref, h_ref, dv_ref, b_ref, bt_ref,
                wmu_ref, bmu_ref, wlv_ref, blv_ref, ge_ref, mu_ref, lv_ref,
                sums, cnts):
    i = pl.program_id(0)
    acc = _join_halves(a_ref[...] + h_ref[...])
    geb = acc * dv_ref[...] + b_ref[...]
    ge_ref[...] = geb
    act = jnp.maximum(geb, 0.0)
    b = bt_ref[0]
    gid = lax.broadcasted_iota(jnp.int32, (G, BR), 0)
    p = (b == gid).astype(jnp.float32)
    ps = lax.dot_general(p, act, (((1,), (0,)), ((), ())),
                         preferred_element_type=jnp.float32)
    pc = lax.dot_general(p, jnp.ones((BR, D), jnp.float32),
                         (((1,), (0,)), ((), ())),
                         preferred_element_type=jnp.float32)

    @pl.when(i == 0)
    def _():
        sums[...] = ps
        cnts[...] = pc

    @pl.when(i > 0)
    def _():
        sums[...] += ps
        cnts[...] += pc

    @pl.when(i == NB - 1)
    def _():
        pooled = sums[...] / jnp.maximum(cnts[...], 1.0)
        mu_ref[...] = lax.dot_general(
            pooled, wmu_ref[...], (((1,), (1,)), ((), ())),
            preferred_element_type=jnp.float32) + bmu_ref[...]
        lv_ref[...] = lax.dot_general(
            pooled, wlv_ref[...], (((1,), (1,)), ((), ())),
            preferred_element_type=jnp.float32) + blv_ref[...]


_final_kernel = pl.pallas_call(
    _final_body,
    grid=(NB,),
    in_specs=[
        _STACK_SPEC,
        _STACK_SPEC,
        pl.BlockSpec((BR, D), lambda i: (i, 0)),
        pl.BlockSpec((1, D), lambda i: (0, 0)),
        pl.BlockSpec((1, 1, BR), lambda i: (i, 0, 0)),
        pl.BlockSpec((L, D), lambda i: (0, 0)),
        pl.BlockSpec((1, L), lambda i: (0, 0)),
        pl.BlockSpec((L, D), lambda i: (0, 0)),
        pl.BlockSpec((1, L), lambda i: (0, 0)),
    ],
    out_specs=[
        pl.BlockSpec((BR, D), lambda i: (i, 0)),
        pl.BlockSpec((G, L), lambda i: (0, 0)),
        pl.BlockSpec((G, L), lambda i: (0, 0)),
    ],
    out_shape=[
        jax.ShapeDtypeStruct((NPAD, D), jnp.float32),
        jax.ShapeDtypeStruct((G, L), jnp.float32),
        jax.ShapeDtypeStruct((G, L), jnp.float32),
    ],
    scratch_shapes=[
        pltpu.VMEM((G, D), jnp.float32),
        pltpu.VMEM((G, D), jnp.float32),
    ],
)


def kernel(x, edge_index, batch, W1, b1, W2, b2, Wmu, bmu, Wlv, blv):
    d_deg = edge_index[1].reshape(NW, NCHD, CD)
    s2 = edge_index[0].reshape(NS, NCHT, CA)
    d2 = edge_index[1].reshape(NS, NCHT, CA)
    xp = jnp.pad(x, ((0, NPAD - N), (0, 0)))
    batch3 = jnp.pad(batch, (0, NPAD - N), constant_values=G).reshape(
        NB, 1, BR)

    deg_kernel, agg_kernel = _sc_kernels()
    cc = deg_kernel(d_deg)
    dinvb = _dinv_kernel(cc[:NPAD].reshape(NPAD // D, D),
                         cc[NPAD:].reshape(NPAD // D, D)).reshape(NPAD, D)

    hp1 = _mm_scale_kernel(xp, W1, dinvb)
    a1 = agg_kernel(hp1.reshape(2 * NPAD, DH), s2, d2).reshape(2, NPAD, DH)
    hp2 = _combine_mm_kernel(a1, hp1, dinvb, b1.reshape(1, D), W2)
    a2 = agg_kernel(hp2.reshape(2 * NPAD, DH), s2, d2).reshape(2, NPAD, DH)
    ge, mu, lv = _final_kernel(a2, hp2, dinvb, b2.reshape(1, D),
                               batch3, Wmu, bmu.reshape(1, L),
                               Wlv, blv.reshape(1, L))
    return mu, lv, ge[:N]


# speedup vs baseline: 29.5525x; 1.3367x over previous
import functools

import jax
import jax.numpy as jnp
from jax import lax
from jax.experimental import pallas as pl
from jax.experimental.pallas import tpu as pltpu
from jax.experimental.pallas import tpu_sc as plsc

N = 10000
E = 320000
D = 128
G = 64
L = 32

NPAD = 10240
NC = 2
NS = 16
NW = NC * NS
EPW = E // NW
CD = 80
NCHD = EPW // CD
EPT = E // NS
CA = 80
NCHT = EPT // CA
NBUF = 5
DH = D // 2
TROWS = NPAD // NS
ZR = 128
BR = 512
NB = NPAD // BR

def _deg_body(d_hbm, out, didx_all, ones_v, zb, acc):
    cid = lax.axis_index("c")
    sid = lax.axis_index("s")
    wid = cid * NS + sid

    @pl.loop(0, CD // 16)
    def _(i):
        ones_v[pl.ds(i * 16, 16)] = jnp.ones((16,), jnp.float32)

    @pl.loop(0, TROWS // 16)
    def _(i):
        zb[pl.ds(i * 16, 16)] = jnp.zeros((16,), jnp.float32)

    pltpu.sync_copy(zb, acc.at[pl.ds(sid * TROWS, TROWS)])
    pltpu.sync_copy(d_hbm.at[wid], didx_all)
    plsc.subcore_barrier()

    @pl.loop(0, NCHD)
    def _(g):
        pltpu.sync_copy(ones_v, acc.at[didx_all.at[g]], add=True)

    plsc.subcore_barrier()
    r = cid * NPAD + sid * TROWS
    pltpu.sync_copy(acc.at[pl.ds(sid * TROWS, TROWS)],
                    out.at[pl.ds(r, TROWS)])


def _agg_body(hp, s_hbm, d_hbm, out, s2v, d2v, zb, acc, *bufs):
    rows = bufs[:NBUF]
    gsem = bufs[NBUF:2 * NBUF]
    ssem = bufs[2 * NBUF:]
    cid = lax.axis_index("c")
    sid = lax.axis_index("s")

    @pl.loop(0, ZR)
    def _(i):
        for j in range(DH // 16):
            zb[i, pl.ds(j * 16, 16)] = jnp.zeros((16,), jnp.float32)

    @pl.loop(0, TROWS // ZR)
    def _(k):
        pltpu.sync_copy(zb, acc.at[pl.ds(sid * TROWS + k * ZR, ZR)])

    pltpu.sync_copy(s_hbm.at[sid], s2v)
    pltpu.sync_copy(d_hbm.at[sid], d2v)
    off = jnp.full((16,), cid * NPAD, jnp.int32)

    @pl.loop(0, NCHT)
    def _(g):
        for j in range(CA // 16):
            s2v[g, pl.ds(j * 16, 16)] = s2v[g, pl.ds(j * 16, 16)] + off

    plsc.subcore_barrier()

    for b in range(NBUF):
        pltpu.async_copy(hp.at[s2v.at[b]], rows[b], gsem[b])

    @pl.loop(0, NCHT, step=NBUF)
    def _(g):
        for b in range(NBUF):
            cur = g + b
            pltpu.make_async_copy(hp.at[s2v.at[cur]], rows[b], gsem[b]).wait()
            pltpu.async_copy(rows[b], acc.at[d2v.at[cur]], ssem[b], add=True)
            pb = (b - 1) % NBUF
            nxt = cur - 1 + NBUF
            cond = (nxt < NCHT) if b > 0 else (g > 0)

            @pl.when(cond)
            def _():
                pltpu.make_async_copy(rows[pb], acc.at[d2v.at[0]],
                                      ssem[pb]).wait()
                pltpu.async_copy(hp.at[s2v.at[nxt]], rows[pb], gsem[pb])

    for b in range(NBUF):
        pltpu.make_async_copy(rows[b], acc.at[d2v.at[0]], ssem[b]).wait()

    plsc.subcore_barrier()

    @pl.loop(0, TROWS // ZR)
    def _(k):
        r = sid * TROWS + k * ZR
        pltpu.sync_copy(acc.at[pl.ds(r, ZR)],
                        out.at[pl.ds(cid * NPAD + r, ZR)])


@functools.lru_cache(maxsize=None)
def _sc_kernels():
    mesh = plsc.VectorSubcoreMesh(core_axis_name="c", subcore_axis_name="s")
    params = pltpu.CompilerParams(use_tc_tiling_on_sc=False)
    deg = functools.partial(
        pl.kernel,
        out_type=jax.ShapeDtypeStruct((2 * NPAD,), jnp.float32),
        mesh=mesh,
        scratch_types=[
            pltpu.VMEM((NCHD, CD), jnp.int32),
            pltpu.VMEM((CD,), jnp.float32),
            pltpu.VMEM((TROWS,), jnp.float32),
            pltpu.VMEM_SHARED((NPAD,), jnp.float32),
        ],
        compiler_params=params,
    )(_deg_body)
    agg = functools.partial(
        pl.kernel,
        out_type=jax.ShapeDtypeStruct((2 * NPAD, DH), jnp.float32),
        mesh=mesh,
        scratch_types=(
            [pltpu.VMEM((NCHT, CA), jnp.int32),
             pltpu.VMEM((NCHT, CA), jnp.int32),
             pltpu.VMEM((ZR, DH), jnp.float32),
             pltpu.VMEM_SHARED((NPAD, DH), jnp.float32)]
            + [pltpu.VMEM((CA, DH), jnp.float32)] * NBUF
            + [pltpu.SemaphoreType.DMA] * (2 * NBUF)
        ),
        compiler_params=params,
    )(_agg_body)
    return deg, agg


def _dinv_body(c0_ref, c1_ref, o_ref):
    deg = 1.0 + c0_ref[...] + c1_ref[...]
    dinv = lax.rsqrt(deg)
    o_ref[...] = jnp.broadcast_to(dinv[:, :, None], (NPAD // D, D, D))


_dinv_kernel = pl.pallas_call(
    _dinv_body,
    out_shape=jax.ShapeDtypeStruct((NPAD // D, D, D), jnp.float32),
)


_STACK_SPEC = pl.BlockSpec((2, BR, DH), lambda i: (0, i, 0))
_STACK_OUT = jax.ShapeDtypeStruct((2, NPAD, DH), jnp.float32)


def _split_halves(o):
    return jnp.stack([o[:, :DH], o[:, DH:]], axis=0)


def _join_halves(s3):
    return jnp.concatenate([s3[0], s3[1]], axis=1)


def _mm_scale_body(x_ref, w_ref, dv_ref, o_ref):
    h = lax.dot_general(x_ref[...], w_ref[...], (((1,), (1,)), ((), ())),
                        preferred_element_type=jnp.float32)
    o_ref[...] = _split_halves(h * dv_ref[...])


_mm_scale_kernel = pl.pallas_call(
    _mm_scale_body,
    grid=(NB,),
    in_specs=[
        pl.BlockSpec((BR, D), lambda i: (i, 0)),
        pl.BlockSpec((D, D), lambda i: (0, 0)),
        pl.BlockSpec((BR, D), lambda i: (i, 0)),
    ],
    out_specs=_STACK_SPEC,
    out_shape=_STACK_OUT,
)


def _combine_mm_body(a_ref, h_ref, dv_ref, b_ref, w_ref, o_ref):
    dv = dv_ref[...]
    acc = _join_halves(a_ref[...] + h_ref[...])
    t = acc * dv + b_ref[...]
    h1 = jnp.maximum(t, 0.0)
    h2 = lax.dot_general(h1, w_ref[...], (((1,), (1,)), ((), ())),
                         preferred_element_type=jnp.float32)
    o_ref[...] = _split_halves(h2 * dv)


_combine_mm_kernel = pl.pallas_call(
    _combine_mm_body,
    grid=(NB,),
    in_specs=[
        _STACK_SPEC,
        _STACK_SPEC,
        pl.BlockSpec((BR, D), lambda i: (i, 0)),
        pl.BlockSpec((1, D), lambda i: (0, 0)),
        pl.BlockSpec((D, D), lambda i: (0, 0)),
    ],
    out_specs=_STACK_SPEC,
    out_shape=_STACK_OUT,
)


def _final_body(a_ref, h_ref, dv_ref, b_ref, bt_ref,
                wmu_ref, bmu_ref, wlv_ref, blv_ref, ge_ref, mu_ref, lv_ref,
                sums, cnts):
    i = pl.program_id(0)
    acc = _join_halves(a_ref[...] + h_ref[...])
    geb = acc * dv_ref[...] + b_ref[...]
    ge_ref[...] = geb
    act = jnp.maximum(geb, 0.0)
    b = bt_ref[0]
    gid = lax.broadcasted_iota(jnp.int32, (G, BR), 0)
    p = (b == gid).astype(jnp.float32)
    ps = lax.dot_general(p, act, (((1,), (0,)), ((), ())),
                         preferred_element_type=jnp.float32)
    pc = lax.dot_general(p, jnp.ones((BR, D), jnp.float32),
                         (((1,), (0,)), ((), ())),
                         preferred_element_type=jnp.float32)

    @pl.when(i == 0)
    def _():
        sums[...] = ps
        cnts[...] = pc

    @pl.when(i > 0)
    def _():
        sums[...] += ps
        cnts[...] += pc

    @pl.when(i == NB - 1)
    def _():
        pooled = sums[...] / jnp.maximum(cnts[...], 1.0)
        mu_ref[...] = lax.dot_general(
            pooled, wmu_ref[...], (((1,), (1,)), ((), ())),
            preferred_element_type=jnp.float32) + bmu_ref[...]
        lv_ref[...] = lax.dot_general(
            pooled, wlv_ref[...], (((1,), (1,)), ((), ())),
            preferred_element_type=jnp.float32) + blv_ref[...]


_final_kernel = pl.pallas_call(
    _final_body,
    grid=(NB,),
    in_specs=[
        _STACK_SPEC,
        _STACK_SPEC,
        pl.BlockSpec((BR, D), lambda i: (i, 0)),
        pl.BlockSpec((1, D), lambda i: (0, 0)),
        pl.BlockSpec((1, 1, BR), lambda i: (i, 0, 0)),
        pl.BlockSpec((L, D), lambda i: (0, 0)),
        pl.BlockSpec((1, L), lambda i: (0, 0)),
        pl.BlockSpec((L, D), lambda i: (0, 0)),
        pl.BlockSpec((1, L), lambda i: (0, 0)),
    ],
    out_specs=[
        pl.BlockSpec((BR, D), lambda i: (i, 0)),
        pl.BlockSpec((G, L), lambda i: (0, 0)),
        pl.BlockSpec((G, L), lambda i: (0, 0)),
    ],
    out_shape=[
        jax.ShapeDtypeStruct((NPAD, D), jnp.float32),
        jax.ShapeDtypeStruct((G, L), jnp.float32),
        jax.ShapeDtypeStruct((G, L), jnp.float32),
    ],
    scratch_shapes=[
        pltpu.VMEM((G, D), jnp.float32),
        pltpu.VMEM((G, D), jnp.float32),
    ],
)


def kernel(x, edge_index, batch, W1, b1, W2, b2, Wmu, bmu, Wlv, blv):
    d_deg = edge_index[1].reshape(NW, NCHD, CD)
    s2 = edge_index[0].reshape(NS, NCHT, CA)
    d2 = edge_index[1].reshape(NS, NCHT, CA)
    xp = jnp.pad(x, ((0, NPAD - N), (0, 0)))
    batch3 = jnp.pad(batch, (0, NPAD - N), constant_values=G).reshape(
        NB, 1, BR)

    deg_kernel, agg_kernel = _sc_kernels()
    cc = deg_kernel(d_deg)
    dinvb = _dinv_kernel(cc[:NPAD].reshape(NPAD // D, D),
                         cc[NPAD:].reshape(NPAD // D, D)).reshape(NPAD, D)

    hp1 = _mm_scale_kernel(xp, W1, dinvb)
    a1 = agg_kernel(hp1.reshape(2 * NPAD, DH), s2, d2).reshape(2, NPAD, DH)
    hp2 = _combine_mm_kernel(a1, hp1, dinvb, b1.reshape(1, D), W2)
    a2 = agg_kernel(hp2.reshape(2 * NPAD, DH), s2, d2).reshape(2, NPAD, DH)
    ge, mu, lv = _final_kernel(a2, hp2, dinvb, b2.reshape(1, D),
                               batch3, Wmu, bmu.reshape(1, L),
                               Wlv, blv.reshape(1, L))
    return mu, lv, ge[:N]
